# baseline (device time: 76283 ns/iter reference)
import jax
import jax.numpy as jnp
from jax import lax
from jax.experimental import pallas as pl
from jax.experimental.pallas import tpu as pltpu

N_Z = 4
N_LANE = 4
S = 4


def kernel(x):
    m, n = x.shape
    m_q = m // N_LANE
    n_s = n // S
    n_h = n // 2

    def body(
        x_ref,
        out_ref,
        pref_in,
        pref_out,
        suff_in,
        suff_out,
        pref_send,
        pref_recv,
        suff_send,
        suff_recv,
        xy_send,
        xy_recv,
    ):
        my_x = lax.axis_index("x")
        my_y = lax.axis_index("y")
        my_z = lax.axis_index("z")
        l = 2 * my_x + my_y
        lx = 2 * (1 - my_x) + my_y
        ly = 2 * my_x + (1 - my_y)
        xdev = (1 - my_x, my_y, my_z)
        ydev = (my_x, 1 - my_y, my_z)

        barrier_sem = pltpu.get_barrier_semaphore()

        def bsig(dev):
            pl.semaphore_signal(
                barrier_sem, inc=1, device_id=dev,
                device_id_type=pl.DeviceIdType.MESH,
            )

        bsig(xdev)
        bsig(ydev)

        @pl.when(my_z < N_Z - 1)
        def _():
            bsig((my_x, my_y, my_z + 1))

        @pl.when(my_z > 0)
        def _():
            bsig((my_x, my_y, my_z - 1))

        pl.semaphore_wait(barrier_sem, 2)

        @pl.when(my_z < N_Z - 1)
        def _():
            pl.semaphore_wait(barrier_sem, 1)

        @pl.when(my_z > 0)
        def _():
            pl.semaphore_wait(barrier_sem, 1)

        q_rows = pl.ds(l * m_q, m_q)

        def pref_rdma(s):
            cs = pl.ds(s * n_s, n_s)
            return pltpu.make_async_remote_copy(
                src_ref=pref_out.at[:, cs],
                dst_ref=pref_in.at[:, cs],
                send_sem=pref_send.at[s],
                recv_sem=pref_recv.at[s],
                device_id=(my_x, my_y, my_z + 1),
                device_id_type=pl.DeviceIdType.MESH,
            )

        def suff_rdma(s):
            cs = pl.ds(s * n_s, n_s)
            return pltpu.make_async_remote_copy(
                src_ref=suff_out.at[:, cs],
                dst_ref=suff_in.at[:, cs],
                send_sem=suff_send.at[s],
                recv_sem=suff_recv.at[s],
                device_id=(my_x, my_y, my_z - 1),
                device_id_type=pl.DeviceIdType.MESH,
            )

        def prefix_block(s):
            cs = pl.ds(s * n_s, n_s)

            @pl.when(my_z == 0)
            def _():
                pref_out[:, cs] = x_ref[q_rows, cs]

            @pl.when(my_z > 0)
            def _():
                pref_rdma(s).wait_recv()
                pref_out[:, cs] = pref_in[:, cs] + x_ref[q_rows, cs]

            @pl.when(my_z < N_Z - 1)
            def _():
                pref_rdma(s).start()

        def suffix_block(s):
            cs = pl.ds(s * n_s, n_s)

            @pl.when(my_z == N_Z - 1)
            def _():
                suff_out[:, cs] = x_ref[q_rows, cs]

            @pl.when(my_z < N_Z - 1)
            def _():
                suff_rdma(s).wait_recv()
                suff_out[:, cs] = suff_in[:, cs] + x_ref[q_rows, cs]

            @pl.when(my_z > 0)
            def _():
                suff_rdma(s).start()

        def z_group(segs):
            @pl.when(my_z <= 1)
            def _():
                for s in segs:
                    prefix_block(s)
                    suffix_block(s)

            @pl.when(my_z >= 2)
            def _():
                for s in segs:
                    suffix_block(s)
                    prefix_block(s)

        def z_total(col_lo):
            cs = pl.ds(col_lo, n_h)

            @pl.when(my_z < N_Z - 1)
            def _():
                out_ref[q_rows, cs] = pref_out[:, cs] + suff_in[:, cs]

            @pl.when(my_z == N_Z - 1)
            def _():
                out_ref[q_rows, cs] = pref_out[:, cs]

        def xy_rdma(lane, col_lo, dev, idx):
            sl = (pl.ds(lane * m_q, m_q), pl.ds(col_lo, n_h))
            return pltpu.make_async_remote_copy(
                src_ref=out_ref.at[sl],
                dst_ref=out_ref.at[sl],
                send_sem=xy_send.at[idx],
                recv_sem=xy_recv.at[idx],
                device_id=dev,
                device_id_type=pl.DeviceIdType.MESH,
            )

        step1A = xy_rdma(l, 0, xdev, 0)
        step1B = xy_rdma(l, n_h, ydev, 1)
        s2A0 = xy_rdma(l, 0, ydev, 2)
        s2A1 = xy_rdma(lx, 0, ydev, 3)
        s2B0 = xy_rdma(l, n_h, xdev, 4)
        s2B1 = xy_rdma(ly, n_h, xdev, 5)

        z_group([0, 1])
        z_total(0)
        step1A.start()
        s2A0.start()

        z_group([2, 3])
        z_total(n_h)
        step1B.start()
        s2B0.start()

        step1A.wait_recv()
        s2A1.start()
        step1B.wait_recv()
        s2B1.start()

        s2A0.wait_recv()
        s2B0.wait_recv()
        s2A1.wait_recv()
        s2B1.wait_recv()

        for d in (step1A, step1B, s2A0, s2A1, s2B0, s2B1):
            d.wait_send()

        @pl.when(my_z < N_Z - 1)
        def _():
            for s in range(S):
                pref_rdma(s).wait_send()

        @pl.when(my_z > 0)
        def _():
            for s in range(S):
                suff_rdma(s).wait_send()

    return pl.pallas_call(
        body,
        out_shape=jax.ShapeDtypeStruct((m, n), jnp.float32),
        in_specs=[pl.BlockSpec(memory_space=pltpu.VMEM)],
        out_specs=pl.BlockSpec(memory_space=pltpu.VMEM),
        scratch_shapes=[
            pltpu.VMEM((m_q, n), jnp.float32),
            pltpu.VMEM((m_q, n), jnp.float32),
            pltpu.VMEM((m_q, n), jnp.float32),
            pltpu.VMEM((m_q, n), jnp.float32),
            pltpu.SemaphoreType.DMA((S,)),
            pltpu.SemaphoreType.DMA((S,)),
            pltpu.SemaphoreType.DMA((S,)),
            pltpu.SemaphoreType.DMA((S,)),
            pltpu.SemaphoreType.DMA((6,)),
            pltpu.SemaphoreType.DMA((6,)),
        ],
        compiler_params=pltpu.CompilerParams(collective_id=0),
    )(x)


# device time: 65662 ns/iter; 1.1618x vs baseline; 1.1618x over previous
import jax
import jax.numpy as jnp
from jax import lax
from jax.experimental import pallas as pl
from jax.experimental.pallas import tpu as pltpu

N_Z = 4
N_LANE = 4
S = 2


def kernel(x):
    m, n = x.shape
    m_q = m // N_LANE
    n_s = n // S
    n_h = n // 2

    def body(
        x_ref,
        out_ref,
        pref_in,
        pref_out,
        suff_in,
        suff_out,
        pref_send,
        pref_recv,
        suff_send,
        suff_recv,
        xy_send,
        xy_recv,
    ):
        my_x = lax.axis_index("x")
        my_y = lax.axis_index("y")
        my_z = lax.axis_index("z")
        l = 2 * my_x + my_y
        lx = 2 * (1 - my_x) + my_y
        ly = 2 * my_x + (1 - my_y)
        xdev = (1 - my_x, my_y, my_z)
        ydev = (my_x, 1 - my_y, my_z)

        barrier_sem = pltpu.get_barrier_semaphore()

        def bsig(dev):
            pl.semaphore_signal(
                barrier_sem, inc=1, device_id=dev,
                device_id_type=pl.DeviceIdType.MESH,
            )

        bsig(xdev)
        bsig(ydev)

        @pl.when(my_z < N_Z - 1)
        def _():
            bsig((my_x, my_y, my_z + 1))

        @pl.when(my_z > 0)
        def _():
            bsig((my_x, my_y, my_z - 1))

        pl.semaphore_wait(barrier_sem, 2)

        @pl.when(my_z < N_Z - 1)
        def _():
            pl.semaphore_wait(barrier_sem, 1)

        @pl.when(my_z > 0)
        def _():
            pl.semaphore_wait(barrier_sem, 1)

        q_rows = pl.ds(l * m_q, m_q)

        def pref_rdma(s):
            cs = pl.ds(s * n_s, n_s)
            return pltpu.make_async_remote_copy(
                src_ref=pref_out.at[:, cs],
                dst_ref=pref_in.at[:, cs],
                send_sem=pref_send.at[s],
                recv_sem=pref_recv.at[s],
                device_id=(my_x, my_y, my_z + 1),
                device_id_type=pl.DeviceIdType.MESH,
            )

        def suff_rdma(s):
            cs = pl.ds(s * n_s, n_s)
            return pltpu.make_async_remote_copy(
                src_ref=suff_out.at[:, cs],
                dst_ref=suff_in.at[:, cs],
                send_sem=suff_send.at[s],
                recv_sem=suff_recv.at[s],
                device_id=(my_x, my_y, my_z - 1),
                device_id_type=pl.DeviceIdType.MESH,
            )

        def prefix_block(s):
            cs = pl.ds(s * n_s, n_s)

            @pl.when(my_z == 0)
            def _():
                pref_out[:, cs] = x_ref[q_rows, cs]

            @pl.when(my_z > 0)
            def _():
                pref_rdma(s).wait_recv()
                pref_out[:, cs] = pref_in[:, cs] + x_ref[q_rows, cs]

            @pl.when(my_z < N_Z - 1)
            def _():
                pref_rdma(s).start()

        def suffix_block(s):
            cs = pl.ds(s * n_s, n_s)

            @pl.when(my_z == N_Z - 1)
            def _():
                suff_out[:, cs] = x_ref[q_rows, cs]

            @pl.when(my_z < N_Z - 1)
            def _():
                suff_rdma(s).wait_recv()
                suff_out[:, cs] = suff_in[:, cs] + x_ref[q_rows, cs]

            @pl.when(my_z > 0)
            def _():
                suff_rdma(s).start()

        def z_group(segs):
            @pl.when(my_z <= 1)
            def _():
                for s in segs:
                    prefix_block(s)
                    suffix_block(s)

            @pl.when(my_z >= 2)
            def _():
                for s in segs:
                    suffix_block(s)
                    prefix_block(s)

        def z_total(col_lo):
            cs = pl.ds(col_lo, n_h)

            @pl.when(my_z < N_Z - 1)
            def _():
                out_ref[q_rows, cs] = pref_out[:, cs] + suff_in[:, cs]

            @pl.when(my_z == N_Z - 1)
            def _():
                out_ref[q_rows, cs] = pref_out[:, cs]

        def xy_rdma(lane, col_lo, dev, idx):
            sl = (pl.ds(lane * m_q, m_q), pl.ds(col_lo, n_h))
            return pltpu.make_async_remote_copy(
                src_ref=out_ref.at[sl],
                dst_ref=out_ref.at[sl],
                send_sem=xy_send.at[idx],
                recv_sem=xy_recv.at[idx],
                device_id=dev,
                device_id_type=pl.DeviceIdType.MESH,
            )

        step1A = xy_rdma(l, 0, xdev, 0)
        step1B = xy_rdma(l, n_h, ydev, 1)
        s2A0 = xy_rdma(l, 0, ydev, 2)
        s2A1 = xy_rdma(lx, 0, ydev, 3)
        s2B0 = xy_rdma(l, n_h, xdev, 4)
        s2B1 = xy_rdma(ly, n_h, xdev, 5)

        z_group([0])
        z_total(0)
        step1A.start()
        s2A0.start()

        z_group([1])
        z_total(n_h)
        step1B.start()
        s2B0.start()

        step1A.wait_recv()
        s2A1.start()
        step1B.wait_recv()
        s2B1.start()

        s2A0.wait_recv()
        s2B0.wait_recv()
        s2A1.wait_recv()
        s2B1.wait_recv()

        for d in (step1A, step1B, s2A0, s2A1, s2B0, s2B1):
            d.wait_send()

        @pl.when(my_z < N_Z - 1)
        def _():
            for s in range(S):
                pref_rdma(s).wait_send()

        @pl.when(my_z > 0)
        def _():
            for s in range(S):
                suff_rdma(s).wait_send()

    return pl.pallas_call(
        body,
        out_shape=jax.ShapeDtypeStruct((m, n), jnp.float32),
        in_specs=[pl.BlockSpec(memory_space=pltpu.VMEM)],
        out_specs=pl.BlockSpec(memory_space=pltpu.VMEM),
        scratch_shapes=[
            pltpu.VMEM((m_q, n), jnp.float32),
            pltpu.VMEM((m_q, n), jnp.float32),
            pltpu.VMEM((m_q, n), jnp.float32),
            pltpu.VMEM((m_q, n), jnp.float32),
            pltpu.SemaphoreType.DMA((S,)),
            pltpu.SemaphoreType.DMA((S,)),
            pltpu.SemaphoreType.DMA((S,)),
            pltpu.SemaphoreType.DMA((S,)),
            pltpu.SemaphoreType.DMA((6,)),
            pltpu.SemaphoreType.DMA((6,)),
        ],
        compiler_params=pltpu.CompilerParams(collective_id=0),
    )(x)
